# baseline (device time: 187292 ns/iter reference)
import jax
import jax.numpy as jnp
from jax import lax
from jax.experimental import pallas as pl
from jax.experimental.pallas import tpu as pltpu

N_DEV = 4
W_TILE = 256
PIECE = 256


def _gelu(y):
    c = 0.7978845608028654
    return 0.5 * y * (1.0 + jnp.tanh(c * (y + 0.044715 * y * y * y)))


def kernel(x, w_mat):
    m_per, k = x.shape
    _, n_per = w_mat.shape
    half = m_per // 2
    n_wtiles = k // W_TILE

    def body(x_ref, w_hbm, out_hbm, comm_ref, wbf_ref, wstage_ref,
             ostage_ref, send_sems, recv_sems, wload_sem, ocopy_sems):
        my = lax.axis_index("i")
        left = lax.rem(my - 1 + N_DEV, N_DEV)
        right = lax.rem(my + 1, N_DEV)

        barrier_sem = pltpu.get_barrier_semaphore()
        for nbr in (left, right):
            pl.semaphore_signal(
                barrier_sem, inc=1,
                device_id=(nbr,), device_id_type=pl.DeviceIdType.MESH,
            )
        pl.semaphore_wait(barrier_sem, 2)

        def fill(q, _):
            p = q // 2
            d = lax.rem(q, 2)
            comm_ref[0, p, d] = (
                x_ref[pl.ds(d * half + p * PIECE, PIECE), :]
                .astype(jnp.bfloat16)
            )
            return 0

        lax.fori_loop(0, 4, fill, 0)

        def make_rdma(slot, nslot, p, d):
            dst_dev = lax.select(d == 0, right, left)
            return pltpu.make_async_remote_copy(
                src_ref=comm_ref.at[slot, p, d],
                dst_ref=comm_ref.at[nslot, p, d],
                send_sem=send_sems.at[slot, p, d],
                recv_sem=recv_sems.at[nslot, p, d],
                device_id=(dst_dev,),
                device_id_type=pl.DeviceIdType.MESH,
            )

        def origin_of(h, d):
            return lax.rem(my + (2 * d - 1) * h + 2 * N_DEV, N_DEV)

        def out_copy(q, origin, d, p):
            return pltpu.make_async_copy(
                ostage_ref.at[pl.ds(q * PIECE, PIECE)],
                out_hbm.at[pl.ds(origin * m_per + d * half + p * PIECE,
                                 PIECE)],
                ocopy_sems.at[q],
            )

        def step(h, _):
            slot = lax.rem(h, 2)
            nslot = lax.rem(h + 1, 2)

            @pl.when(h < N_DEV - 1)
            def _():
                def launch(i, _):
                    make_rdma(slot, nslot, i // 2, lax.rem(i, 2)).start()
                    return 0

                lax.fori_loop(0, 4, launch, 0)

            @pl.when(h == 0)
            def _():
                def wconv(kt, _):
                    cp = pltpu.make_async_copy(
                        w_hbm.at[pl.ds(kt * W_TILE, W_TILE)],
                        wstage_ref, wload_sem,
                    )
                    cp.start()
                    cp.wait()
                    wbf_ref[pl.ds(kt * W_TILE, W_TILE), :] = (
                        wstage_ref[...].astype(jnp.bfloat16)
                    )
                    return 0

                lax.fori_loop(0, n_wtiles, wconv, 0)

            def phase_a(_):
                @pl.when(h > 0)
                def _():
                    out_copy(0, origin_of(h - 1, 0), 0, 1).wait()
                    out_copy(1, origin_of(h - 1, 1), 1, 1).wait()

                acc = jnp.dot(
                    comm_ref[slot, 1].reshape(2 * PIECE, k),
                    wbf_ref[...],
                    preferred_element_type=jnp.float32,
                )
                ostage_ref[pl.ds(0, 2 * PIECE), :] = (
                    _gelu(acc).astype(jnp.bfloat16)
                )
                out_copy(0, origin_of(h, 0), 0, 1).start()
                out_copy(1, origin_of(h, 1), 1, 1).start()

            phase_a(None)

            @pl.when(h < N_DEV - 1)
            def _():
                make_rdma(slot, nslot, 0, 0).wait()
                make_rdma(slot, nslot, 0, 1).wait()

                def phase_b(j, _):
                    src_slot = lax.select(j == 0, nslot, slot)
                    ch = lax.select(j == 0, h + 1, 0)

                    @pl.when(jnp.logical_or(h > 0, j > 0))
                    def _():
                        prev_ch = lax.select(j == 0, h, h + 1)
                        out_copy(2, origin_of(prev_ch, 0), 0, 0).wait()
                        out_copy(3, origin_of(prev_ch, 1), 1, 0).wait()

                    acc = jnp.dot(
                        comm_ref[src_slot, 0].reshape(2 * PIECE, k),
                        wbf_ref[...],
                        preferred_element_type=jnp.float32,
                    )
                    ostage_ref[pl.ds(2 * PIECE, 2 * PIECE), :] = (
                        _gelu(acc).astype(jnp.bfloat16)
                    )
                    out_copy(2, origin_of(ch, 0), 0, 0).start()
                    out_copy(3, origin_of(ch, 1), 1, 0).start()
                    return 0

                n_b = lax.select(h == 0, 2, 1)
                lax.fori_loop(0, n_b, phase_b, 0)

                make_rdma(slot, nslot, 1, 0).wait()
                make_rdma(slot, nslot, 1, 1).wait()

            return 0

        lax.fori_loop(0, N_DEV, step, 0)

        out_copy(0, origin_of(N_DEV - 1, 0), 0, 1).wait()
        out_copy(1, origin_of(N_DEV - 1, 1), 1, 1).wait()
        out_copy(2, origin_of(N_DEV - 1, 0), 0, 0).wait()
        out_copy(3, origin_of(N_DEV - 1, 1), 1, 0).wait()

    out = pl.pallas_call(
        body,
        out_shape=jax.ShapeDtypeStruct((N_DEV * m_per, n_per), jnp.bfloat16),
        in_specs=[
            pl.BlockSpec(memory_space=pltpu.VMEM),
            pl.BlockSpec(memory_space=pl.ANY),
        ],
        out_specs=pl.BlockSpec(memory_space=pl.ANY),
        scratch_shapes=[
            pltpu.VMEM((2, 2, 2, PIECE, k), jnp.bfloat16),
            pltpu.VMEM((k, n_per), jnp.bfloat16),
            pltpu.VMEM((W_TILE, n_per), jnp.float32),
            pltpu.VMEM((m_per, n_per), jnp.bfloat16),
            pltpu.SemaphoreType.DMA((2, 2, 2)),
            pltpu.SemaphoreType.DMA((2, 2, 2)),
            pltpu.SemaphoreType.DMA,
            pltpu.SemaphoreType.DMA((4,)),
        ],
        compiler_params=pltpu.CompilerParams(
            collective_id=0,
            vmem_limit_bytes=63 * 1024 * 1024,
        ),
    )(x, w_mat)
    return out


# device time: 178291 ns/iter; 1.0505x vs baseline; 1.0505x over previous
import jax
import jax.numpy as jnp
from jax import lax
from jax.experimental import pallas as pl
from jax.experimental.pallas import tpu as pltpu

N_DEV = 4
W_TILE = 64
W_DEPTH = 4
PIECE = 256


def _gelu(y):
    c = 0.7978845608028654
    return 0.5 * y * (1.0 + jnp.tanh(c * (y + 0.044715 * y * y * y)))


def kernel(x, w_mat):
    m_per, k = x.shape
    _, n_per = w_mat.shape
    half = m_per // 2
    n_wtiles = k // W_TILE

    def body(x_ref, w_hbm, out_hbm, comm_ref, wbf_ref, wstage_ref,
             ostage_ref, send_sems, recv_sems, wload_sems, ocopy_sems):
        my = lax.axis_index("i")
        left = lax.rem(my - 1 + N_DEV, N_DEV)
        right = lax.rem(my + 1, N_DEV)

        def w_tile_copy(kt):
            return pltpu.make_async_copy(
                w_hbm.at[pl.ds(kt * W_TILE, W_TILE)],
                wstage_ref.at[lax.rem(kt, W_DEPTH)],
                wload_sems.at[lax.rem(kt, W_DEPTH)],
            )

        def wkick(kt, _):
            w_tile_copy(kt).start()
            return 0

        lax.fori_loop(0, W_DEPTH, wkick, 0)

        barrier_sem = pltpu.get_barrier_semaphore()
        for nbr in (left, right):
            pl.semaphore_signal(
                barrier_sem, inc=1,
                device_id=(nbr,), device_id_type=pl.DeviceIdType.MESH,
            )
        pl.semaphore_wait(barrier_sem, 2)

        def make_rdma(slot, nslot, p, d):
            dst_dev = lax.select(d == 0, right, left)
            return pltpu.make_async_remote_copy(
                src_ref=comm_ref.at[slot, p, d],
                dst_ref=comm_ref.at[nslot, p, d],
                send_sem=send_sems.at[slot, p, d],
                recv_sem=recv_sems.at[nslot, p, d],
                device_id=(dst_dev,),
                device_id_type=pl.DeviceIdType.MESH,
            )

        def origin_of(h, d):
            return lax.rem(my + (2 * d - 1) * h + 2 * N_DEV, N_DEV)

        def out_copy(q, origin, d, p):
            return pltpu.make_async_copy(
                ostage_ref.at[pl.ds(q * PIECE, PIECE)],
                out_hbm.at[pl.ds(origin * m_per + d * half + p * PIECE,
                                 PIECE)],
                ocopy_sems.at[q],
            )

        def fill(q, _):
            p = q // 2
            d = lax.rem(q, 2)
            comm_ref[0, p, d] = (
                x_ref[pl.ds(d * half + p * PIECE, PIECE), :]
                .astype(jnp.bfloat16)
            )
            make_rdma(0, 1, p, d).start()
            return 0

        lax.fori_loop(0, 4, fill, 0)

        def step(h, _):
            slot = lax.rem(h, 2)
            nslot = lax.rem(h + 1, 2)

            @pl.when(jnp.logical_and(h > 0, h < N_DEV - 1))
            def _():
                def launch(i, _):
                    make_rdma(slot, nslot, i // 2, lax.rem(i, 2)).start()
                    return 0

                lax.fori_loop(0, 4, launch, 0)

            @pl.when(h == 0)
            def _():
                def wconv(kt, _):
                    w_tile_copy(kt).wait()
                    wbf_ref[pl.ds(kt * W_TILE, W_TILE), :] = (
                        wstage_ref[lax.rem(kt, W_DEPTH)].astype(jnp.bfloat16)
                    )

                    @pl.when(kt + W_DEPTH < n_wtiles)
                    def _():
                        w_tile_copy(kt + W_DEPTH).start()

                    return 0

                lax.fori_loop(0, n_wtiles, wconv, 0)

            def phase_a(_):
                @pl.when(h > 0)
                def _():
                    out_copy(0, origin_of(h - 1, 0), 0, 1).wait()
                    out_copy(1, origin_of(h - 1, 1), 1, 1).wait()

                acc = jnp.dot(
                    comm_ref[slot, 1].reshape(2 * PIECE, k),
                    wbf_ref[...],
                    preferred_element_type=jnp.float32,
                )
                ostage_ref[pl.ds(0, 2 * PIECE), :] = (
                    _gelu(acc).astype(jnp.bfloat16)
                )
                out_copy(0, origin_of(h, 0), 0, 1).start()
                out_copy(1, origin_of(h, 1), 1, 1).start()

            phase_a(None)

            @pl.when(h < N_DEV - 1)
            def _():
                make_rdma(slot, nslot, 0, 0).wait()
                make_rdma(slot, nslot, 0, 1).wait()

                def phase_b(j, _):
                    src_slot = lax.select(j == 0, nslot, slot)
                    ch = lax.select(j == 0, h + 1, 0)

                    @pl.when(jnp.logical_or(h > 0, j > 0))
                    def _():
                        prev_ch = lax.select(j == 0, h, h + 1)
                        out_copy(2, origin_of(prev_ch, 0), 0, 0).wait()
                        out_copy(3, origin_of(prev_ch, 1), 1, 0).wait()

                    acc = jnp.dot(
                        comm_ref[src_slot, 0].reshape(2 * PIECE, k),
                        wbf_ref[...],
                        preferred_element_type=jnp.float32,
                    )
                    ostage_ref[pl.ds(2 * PIECE, 2 * PIECE), :] = (
                        _gelu(acc).astype(jnp.bfloat16)
                    )
                    out_copy(2, origin_of(ch, 0), 0, 0).start()
                    out_copy(3, origin_of(ch, 1), 1, 0).start()
                    return 0

                n_b = lax.select(h == 0, 2, 1)
                lax.fori_loop(0, n_b, phase_b, 0)

                make_rdma(slot, nslot, 1, 0).wait()
                make_rdma(slot, nslot, 1, 1).wait()

            return 0

        lax.fori_loop(0, N_DEV, step, 0)

        out_copy(0, origin_of(N_DEV - 1, 0), 0, 1).wait()
        out_copy(1, origin_of(N_DEV - 1, 1), 1, 1).wait()
        out_copy(2, origin_of(N_DEV - 1, 0), 0, 0).wait()
        out_copy(3, origin_of(N_DEV - 1, 1), 1, 0).wait()

    out = pl.pallas_call(
        body,
        out_shape=jax.ShapeDtypeStruct((N_DEV * m_per, n_per), jnp.bfloat16),
        in_specs=[
            pl.BlockSpec(memory_space=pltpu.VMEM),
            pl.BlockSpec(memory_space=pl.ANY),
        ],
        out_specs=pl.BlockSpec(memory_space=pl.ANY),
        scratch_shapes=[
            pltpu.VMEM((2, 2, 2, PIECE, k), jnp.bfloat16),
            pltpu.VMEM((k, n_per), jnp.bfloat16),
            pltpu.VMEM((W_DEPTH, W_TILE, n_per), jnp.float32),
            pltpu.VMEM((m_per, n_per), jnp.bfloat16),
            pltpu.SemaphoreType.DMA((2, 2, 2)),
            pltpu.SemaphoreType.DMA((2, 2, 2)),
            pltpu.SemaphoreType.DMA((W_DEPTH,)),
            pltpu.SemaphoreType.DMA((4,)),
        ],
        compiler_params=pltpu.CompilerParams(
            collective_id=0,
            vmem_limit_bytes=63 * 1024 * 1024,
        ),
    )(x, w_mat)
    return out
